# baseline (speedup 1.0000x reference)
import functools

import jax
import jax.numpy as jnp
from jax import lax
from jax.experimental import pallas as pl
from jax.experimental.pallas import tpu as pltpu
from jax.experimental.pallas import tpu_sc as plsc

_VOCAB = 1000000
_EMBED = 32
_BATCH = 16384
_HIST = 200

_NC = 2
_NS = 16
_NW = _NC * _NS
_BT_PER_W = (_BATCH // 128) // _NW
_HC = 8
_NH = _HIST // _HC
_STEPS = _BT_PER_W * _NH

_WIDEN_BLK = 8192
_WIDEN_GRID = -(-_VOCAB // _WIDEN_BLK)
_WIDE_ROWS = _WIDEN_GRID * _WIDEN_BLK


def _widen_body(t_ref, o_ref):
  t = jnp.transpose(t_ref[...])
  o_ref[...] = jnp.pad(t, ((0, 0), (0, 128 - _EMBED)))


_widen = pl.pallas_call(
    _widen_body,
    out_shape=jax.ShapeDtypeStruct((_WIDE_ROWS, 128), jnp.float32),
    grid=(_WIDEN_GRID,),
    in_specs=[pl.BlockSpec((_EMBED, _WIDEN_BLK), lambda i: (0, i))],
    out_specs=pl.BlockSpec((_WIDEN_BLK, 128), lambda i: (i, 0)),
)


def _gather_body(idxT_hbm, packed_hbm, out_hbm, idx_v,
                 f0, f1, ov0, ov1, gs0, gs1, os0, os1):
  wid = lax.axis_index("s") * _NC + lax.axis_index("c")
  fbufs = (f0, f1)
  gsems = (gs0, gs1)
  lanes = lax.iota(jnp.int32, 16)

  def fire(gidx_row, fp):
    pltpu.async_copy(packed_hbm.at[gidx_row], fbufs[fp], gsems[fp])

  def drain(fp):
    pltpu.make_async_copy(
        packed_hbm.at[idx_v.at[0]], fbufs[fp], gsems[fp]).wait()

  def do_step(i, ov, osem):
    bt = i // _NH
    hc = i % _NH
    h0 = hc * _HC
    b0 = (wid * _BT_PER_W + bt) * 128

    pltpu.sync_copy(idxT_hbm.at[pl.ds(h0, _HC), pl.ds(b0, 128)], idx_v)

    @pl.when(i >= 2)
    def _():
      pltpu.make_async_copy(
          ov, out_hbm.at[pl.ds(0, _HC), :, pl.ds(0, 128)], osem).wait()

    fire(idx_v.at[0], 0)
    fire(idx_v.at[1], 1)
    for h in range(_HC):
      fp = h % 2
      drain(fp)

      def assemble(j, carry):
        rows = lanes + j * 16
        for e in range(_EMBED):
          ov[h, e, pl.ds(j * 16, 16)] = plsc.load_gather(
              fbufs[fp], [rows, jnp.full((16,), e, jnp.int32)])
        return carry

      lax.fori_loop(0, 8, assemble, 0)
      if h + 2 < _HC:
        fire(idx_v.at[h + 2], fp)

    pltpu.async_copy(
        ov, out_hbm.at[pl.ds(h0, _HC), :, pl.ds(b0, 128)], osem)

  def pair(g, carry):
    do_step(g * 2, ov0, os0)
    do_step(g * 2 + 1, ov1, os1)
    return carry

  lax.fori_loop(0, _STEPS // 2, pair, 0)
  pltpu.make_async_copy(
      ov0, out_hbm.at[pl.ds(0, _HC), :, pl.ds(0, 128)], os0).wait()
  pltpu.make_async_copy(
      ov1, out_hbm.at[pl.ds(0, _HC), :, pl.ds(0, 128)], os1).wait()


_gather = functools.partial(
    pl.kernel,
    out_type=jax.ShapeDtypeStruct((_HIST, _EMBED, _BATCH), jnp.float32),
    mesh=plsc.VectorSubcoreMesh(core_axis_name="c", subcore_axis_name="s"),
    scratch_types=[
        pltpu.VMEM((_HC, 128), jnp.int32),
        pltpu.VMEM((128, 128), jnp.float32),
        pltpu.VMEM((128, 128), jnp.float32),
        pltpu.VMEM((_HC, _EMBED, 128), jnp.float32),
        pltpu.VMEM((_HC, _EMBED, 128), jnp.float32),
        pltpu.SemaphoreType.DMA,
        pltpu.SemaphoreType.DMA,
        pltpu.SemaphoreType.DMA,
        pltpu.SemaphoreType.DMA,
    ],
    compiler_params=pltpu.CompilerParams(
        use_tc_tiling_on_sc=True, needs_layout_passes=False),
)(_gather_body)


def _mask_body(x_ref, o_ref):
  o_ref[...] = x_ref[...] != 0


_mask = pl.pallas_call(
    _mask_body,
    out_shape=jax.ShapeDtypeStruct((_HIST, _BATCH), jnp.bool_),
    grid=(16,),
    in_specs=[pl.BlockSpec((_HIST, _BATCH // 16), lambda i: (0, i))],
    out_specs=pl.BlockSpec((_HIST, _BATCH // 16), lambda i: (0, i)),
)


@jax.jit
def kernel(inputs, table):
  inputsT = jnp.transpose(inputs)
  packed = _widen(jnp.transpose(table))
  out = _gather(inputsT, packed)
  maskT = _mask(inputsT)
  return jnp.transpose(out, (2, 0, 1)), jnp.transpose(maskT)


# speedup vs baseline: 1.2102x; 1.2102x over previous
import functools

import jax
import jax.numpy as jnp
from jax import lax
from jax.experimental import pallas as pl
from jax.experimental.pallas import tpu as pltpu
from jax.experimental.pallas import tpu_sc as plsc

_VOCAB = 1000000
_EMBED = 32
_BATCH = 16384
_HIST = 200

_NC = 2
_NS = 16
_NW = _NC * _NS
_RPW = _BATCH // _NW
_RC = 8
_NSTEPS = _RPW // _RC


def _gather_body(idx_hbm, table_hbm, out_hbm, idx_v, rows_v, gsem, osem0,
                 osem1):
  wid = lax.axis_index("s") * _NC + lax.axis_index("c")
  base = wid * _RPW
  osems = (osem0, osem1)

  def do_step(i, b):
    row0 = base + i * _RC
    pltpu.sync_copy(idx_hbm.at[pl.ds(row0, _RC), :], idx_v.at[b])

    @pl.when(i >= 2)
    def _():
      pltpu.make_async_copy(
          rows_v.at[b], out_hbm.at[pl.ds(row0, _RC)], osems[b]).wait()

    for j in range(_RC):
      pltpu.async_copy(table_hbm.at[idx_v.at[b, j]], rows_v.at[b, j], gsem)
    for j in range(_RC):
      pltpu.make_async_copy(
          table_hbm.at[idx_v.at[b, j]], rows_v.at[b, j], gsem).wait()

    pltpu.async_copy(rows_v.at[b], out_hbm.at[pl.ds(row0, _RC)], osems[b])

  def pair(g, carry):
    do_step(g * 2, 0)
    do_step(g * 2 + 1, 1)
    return carry

  lax.fori_loop(0, _NSTEPS // 2, pair, 0)
  pltpu.make_async_copy(
      rows_v.at[0], out_hbm.at[pl.ds(base, _RC)], osem0).wait()
  pltpu.make_async_copy(
      rows_v.at[1], out_hbm.at[pl.ds(base, _RC)], osem1).wait()


_gather = functools.partial(
    pl.kernel,
    out_type=jax.ShapeDtypeStruct((_BATCH, _HIST, _EMBED), jnp.float32),
    mesh=plsc.VectorSubcoreMesh(core_axis_name="c", subcore_axis_name="s"),
    scratch_types=[
        pltpu.VMEM((2, _RC, _HIST), jnp.int32),
        pltpu.VMEM((2, _RC, _HIST, _EMBED), jnp.float32),
        pltpu.SemaphoreType.DMA,
        pltpu.SemaphoreType.DMA,
        pltpu.SemaphoreType.DMA,
    ],
    compiler_params=pltpu.CompilerParams(use_tc_tiling_on_sc=False),
)(_gather_body)


def _mask_body(x_ref, o_ref):
  o_ref[...] = x_ref[...] != 0


_mask = pl.pallas_call(
    _mask_body,
    out_shape=jax.ShapeDtypeStruct((_BATCH, _HIST), jnp.bool_),
    grid=(16,),
    in_specs=[pl.BlockSpec((_BATCH // 16, _HIST), lambda i: (i, 0))],
    out_specs=pl.BlockSpec((_BATCH // 16, _HIST), lambda i: (i, 0)),
)


@jax.jit
def kernel(inputs, table):
  rows = _gather(inputs, table)
  mask = _mask(inputs)
  return rows, mask
